# Initial kernel scaffold; baseline (speedup 1.0000x reference)
import jax
import jax.numpy as jnp
from jax.experimental import pallas as pl


def kernel(idx, emb1, emb2, W1, b1, W2, b2):
    raise NotImplementedError("write your pallas kernel here")


# speedup vs baseline: 13.1910x; 13.1910x over previous
import jax
import jax.numpy as jnp
from jax.experimental import pallas as pl
from jax.experimental.pallas import tpu as pltpu

N = 8192
D = 64
KSEL = 64
ALPHA = 3.0
RBLK = 128
NBLK = N // RBLK
CH = 64
LW = N // CH


def _nv_kernel(e1_ref, e2_ref, w1_ref, b1_ref, w2_ref, b2_ref, o1_ref, o2_ref):
    dn = (((1,), (1,)), ((), ()))
    x1 = jax.lax.dot_general(e1_ref[...], w1_ref[...], dn,
                             preferred_element_type=jnp.float32)
    o1_ref[...] = jnp.tanh(ALPHA * (x1 + b1_ref[...]))
    x2 = jax.lax.dot_general(e2_ref[...], w2_ref[...], dn,
                             preferred_element_type=jnp.float32)
    o2_ref[...] = jnp.tanh(ALPHA * (x2 + b2_ref[...]))


def _adj_kernel(nv1b_ref, nv2b_ref, nv1_ref, nv2_ref, noise_ref, out_ref):
    dn = (((1,), (1,)), ((), ()))
    a = jax.lax.dot_general(nv1b_ref[...], nv2_ref[...], dn,
                            preferred_element_type=jnp.float32)
    a -= jax.lax.dot_general(nv2b_ref[...], nv1_ref[...], dn,
                             preferred_element_type=jnp.float32)
    adj = jnp.maximum(jnp.tanh(ALPHA * a), 0.0)
    v = adj + noise_ref[...]
    bits = jax.lax.bitcast_convert_type(v, jnp.int32)

    kf = float(KSEL)

    def body(_, carry):
        lo, hi = carry
        mid = jax.lax.shift_right_logical(lo + hi, 1)
        c = jnp.sum(jnp.where(bits >= mid, 1.0, 0.0), axis=1, keepdims=True)
        ge = c >= kf
        return jnp.where(ge, mid, lo), jnp.where(ge, hi, mid)

    lo0 = jnp.zeros((RBLK, 1), jnp.int32)
    hi0 = jnp.full((RBLK, 1), 1 << 30, jnp.int32)
    t, _ = jax.lax.fori_loop(0, 30, body, (lo0, hi0), unroll=True)

    gt = bits > t
    c_gt = jnp.sum(jnp.where(gt, 1.0, 0.0), axis=1, keepdims=True)
    rrem = kf - c_gt
    tie = bits == t
    tie_f = jnp.where(tie, 1.0, 0.0)

    t2 = tie_f.reshape(RBLK * CH, LW)
    jj = jax.lax.broadcasted_iota(jnp.int32, (LW, LW), 0)
    kk = jax.lax.broadcasted_iota(jnp.int32, (LW, LW), 1)
    upper = jnp.where(jj < kk, 1.0, 0.0)
    within = jnp.dot(t2, upper, preferred_element_type=jnp.float32)
    within3 = within.reshape(RBLK, CH, LW)
    ctot = within3[:, :, LW - 1] + tie_f.reshape(RBLK, CH, LW)[:, :, LW - 1]
    jj2 = jax.lax.broadcasted_iota(jnp.int32, (CH, CH), 0)
    kk2 = jax.lax.broadcasted_iota(jnp.int32, (CH, CH), 1)
    upper2 = jnp.where(jj2 < kk2, 1.0, 0.0)
    offs = jnp.dot(ctot, upper2, preferred_element_type=jnp.float32)
    prefix = (within3 + offs[:, :, None]).reshape(RBLK, N)

    mask = gt | (tie & (prefix < rrem))
    out_ref[...] = jnp.where(mask, adj, 0.0)


def _compute_nv(nodevec1, nodevec2, W1, b1, W2, b2):
    return pl.pallas_call(
        _nv_kernel,
        out_shape=(jax.ShapeDtypeStruct((N, D), jnp.float32),
                   jax.ShapeDtypeStruct((N, D), jnp.float32)),
    )(nodevec1, nodevec2, W1, b1.reshape(1, D), W2, b2.reshape(1, D))


def _compute_adj(nv1, nv2, noise):
    return pl.pallas_call(
        _adj_kernel,
        grid=(NBLK,),
        in_specs=[
            pl.BlockSpec((RBLK, D), lambda i: (i, 0)),
            pl.BlockSpec((RBLK, D), lambda i: (i, 0)),
            pl.BlockSpec((N, D), lambda i: (0, 0)),
            pl.BlockSpec((N, D), lambda i: (0, 0)),
            pl.BlockSpec((RBLK, N), lambda i: (i, 0)),
        ],
        out_specs=pl.BlockSpec((RBLK, N), lambda i: (i, 0)),
        out_shape=jax.ShapeDtypeStruct((N, N), jnp.float32),
        compiler_params=pltpu.CompilerParams(
            dimension_semantics=("arbitrary",),
        ),
    )(nv1, nv2, nv1, nv2, noise)


def kernel(idx, emb1, emb2, W1, b1, W2, b2):
    nodevec1 = jnp.take(emb1, idx, axis=0)
    nodevec2 = jnp.take(emb2, idx, axis=0)
    nv1, nv2 = _compute_nv(nodevec1, nodevec2, W1, b1, W2, b2)
    noise = jax.random.uniform(jax.random.key(1234), (N, N),
                               dtype=jnp.float32) * 0.01
    return _compute_adj(nv1, nv2, noise)
